# baseline (speedup 1.0000x reference)
import functools
import math

import jax
import jax.numpy as jnp
from jax import lax
from jax.experimental import pallas as pl
from jax.experimental.pallas import tpu as pltpu
from jax.experimental.pallas import tpu_sc as plsc

_N_NODES = 100000
_N_EDGES = 6400000
_NUM_TYPES = 4
_FACTOR = 1.0 / math.sqrt(64.0)

_LANES = 16
_ROWS = 16
_CHUNK = 128
_BLK = _ROWS * _CHUNK
_NBLK = _N_EDGES // _BLK
_NW = 32
_BASE_BLKS = _NBLK // _NW
_EXTRA = _NBLK - _BASE_BLKS * _NW

_ACC_PAD = 102400
_TILE_SLICE = _ACC_PAD // 16


def _sc_partial_sums(eng3, ei4, species, table16):
    mesh = plsc.VectorSubcoreMesh(core_axis_name="c", subcore_axis_name="s")

    @functools.partial(
        pl.kernel,
        mesh=mesh,
        compiler_params=pltpu.CompilerParams(needs_layout_passes=False),
        out_type=jax.ShapeDtypeStruct((2, _ACC_PAD), jnp.float32),
        scratch_types=[
            pltpu.VMEM((_N_NODES,), jnp.int32),
            pltpu.VMEM((_LANES,), jnp.float32),
            pltpu.VMEM((_BLK,), jnp.int32),
            pltpu.VMEM((_BLK,), jnp.int32),
            pltpu.VMEM((_BLK,), jnp.float32),
            pltpu.VMEM((_BLK,), jnp.float32),
            pltpu.VMEM((_TILE_SLICE,), jnp.float32),
            pltpu.VMEM_SHARED((_ACC_PAD,), jnp.float32),
        ],
    )
    def k(eng_hbm, ei_hbm, species_hbm, table_hbm, out_hbm,
          species_v, table_v, cen_v, nei_v, eng_v, val_v, stage_v, acc_sh):
        cid = lax.axis_index("c")
        tid = lax.axis_index("s")
        wid = tid * 2 + cid

        pltpu.sync_copy(species_hbm, species_v)
        pltpu.sync_copy(table_hbm, table_v)
        table_v[...] = table_v[...] * _FACTOR

        zeros16 = jnp.zeros((_LANES,), jnp.float32)

        def zbody(i, _):
            stage_v[pl.ds(i * _LANES, _LANES)] = zeros16
            return 0

        lax.fori_loop(0, _TILE_SLICE // _LANES, zbody, 0)
        pltpu.sync_copy(stage_v, acc_sh.at[pl.ds(tid * _TILE_SLICE, _TILE_SLICE)])
        plsc.subcore_barrier()

        nblk = jnp.where(wid < _EXTRA, _BASE_BLKS + 1, _BASE_BLKS)
        blk0 = _BASE_BLKS * wid + jnp.minimum(wid, _EXTRA)

        def block_body(b, _):
            blk = blk0 + b
            pltpu.sync_copy(ei_hbm.at[0, blk], cen_v)
            pltpu.sync_copy(ei_hbm.at[1, blk], nei_v)
            pltpu.sync_copy(eng_hbm.at[blk], eng_v)

            def sub_body(q, _):
                s = q * _LANES
                c = cen_v[pl.ds(s, _LANES)]
                n = nei_v[pl.ds(s, _LANES)]
                cs = plsc.load_gather(species_v, [c])
                ns = plsc.load_gather(species_v, [n])
                scale = plsc.load_gather(table_v, [cs * _NUM_TYPES + ns])
                val_v[pl.ds(s, _LANES)] = eng_v[pl.ds(s, _LANES)] * scale
                return 0

            lax.fori_loop(0, _BLK // _LANES, sub_body, 0)

            pltpu.sync_copy(val_v, acc_sh.at[cen_v], add=True)
            return 0

        lax.fori_loop(0, nblk, block_body, 0)
        plsc.subcore_barrier()

        sl = pl.ds(tid * _TILE_SLICE, _TILE_SLICE)
        pltpu.sync_copy(acc_sh.at[sl], stage_v)
        pltpu.sync_copy(stage_v, out_hbm.at[cid, sl])

    return k(eng3, ei4, species, table16)


def _tc_add(partials):
    def body(p_ref, o_ref):
        o_ref[...] = p_ref[0] + p_ref[1]

    return pl.pallas_call(
        body,
        out_shape=jax.ShapeDtypeStruct((_ACC_PAD // 128, 128), jnp.float32),
    )(partials.reshape(2, _ACC_PAD // 128, 128))


def kernel(edge_energy, per_edge_scales, edge_index, atom_types):
    eng3 = edge_energy.reshape(_NBLK, _BLK)
    ei4 = edge_index.reshape(2, _NBLK, _BLK)
    species = atom_types.reshape(_N_NODES)
    table16 = per_edge_scales.reshape(_NUM_TYPES * _NUM_TYPES)

    partials = _sc_partial_sums(eng3, ei4, species, table16)
    summed = _tc_add(partials)
    return summed.reshape(_ACC_PAD)[:_N_NODES].reshape(_N_NODES, 1)


# speedup vs baseline: 2.5412x; 2.5412x over previous
import functools
import math

import jax
import jax.numpy as jnp
from jax import lax
from jax.experimental import pallas as pl
from jax.experimental.pallas import tpu as pltpu
from jax.experimental.pallas import tpu_sc as plsc

_N_NODES = 100000
_N_EDGES = 6400000
_NUM_TYPES = 4
_FACTOR = 1.0 / math.sqrt(64.0)

_LANES = 16
_ROWS = 16
_CHUNK = 128
_BLK = _ROWS * _CHUNK
_NBLK = _N_EDGES // _BLK
_NW = 32
_BASE_BLKS = _NBLK // _NW
_EXTRA = _NBLK - _BASE_BLKS * _NW

_ACC_PAD = 102400
_TILE_SLICE = _ACC_PAD // 16


def _sc_partial_sums(eng3, ei4, species, table16):
    mesh = plsc.VectorSubcoreMesh(core_axis_name="c", subcore_axis_name="s")
    blk_buf = pltpu.VMEM((_ROWS, _CHUNK), jnp.int32)
    blk_buf_f = pltpu.VMEM((_ROWS, _CHUNK), jnp.float32)

    @functools.partial(
        pl.kernel,
        mesh=mesh,
        compiler_params=pltpu.CompilerParams(needs_layout_passes=False),
        out_type=jax.ShapeDtypeStruct((2, _ACC_PAD), jnp.float32),
        scratch_types=[
            pltpu.VMEM((_N_NODES,), jnp.int32),
            pltpu.VMEM((_LANES,), jnp.float32),
            blk_buf, blk_buf, blk_buf_f, blk_buf_f,
            blk_buf, blk_buf, blk_buf_f, blk_buf_f,
            pltpu.VMEM((_TILE_SLICE,), jnp.float32),
            pltpu.VMEM_SHARED((_ACC_PAD,), jnp.float32),
            pltpu.SemaphoreType.DMA,
            pltpu.SemaphoreType.DMA,
            pltpu.SemaphoreType.DMA,
            pltpu.SemaphoreType.DMA,
        ],
    )
    def k(eng_hbm, ei_hbm, species_hbm, table_hbm, out_hbm,
          species_v, table_v,
          cen_a, nei_a, eng_a, val_a,
          cen_b, nei_b, eng_b, val_b,
          stage_v, acc_sh,
          in_sem_a, in_sem_b, scat_sem_a, scat_sem_b):
        cid = lax.axis_index("c")
        tid = lax.axis_index("s")
        wid = tid * 2 + cid

        pltpu.sync_copy(species_hbm, species_v)
        pltpu.sync_copy(table_hbm, table_v)
        table_v[...] = table_v[...] * _FACTOR

        zeros16 = jnp.zeros((_LANES,), jnp.float32)

        def zbody(i, _):
            stage_v[pl.ds(i * _LANES, _LANES)] = zeros16
            return 0

        lax.fori_loop(0, _TILE_SLICE // _LANES, zbody, 0)
        pltpu.sync_copy(stage_v, acc_sh.at[pl.ds(tid * _TILE_SLICE, _TILE_SLICE)])
        plsc.subcore_barrier()

        nblk = jnp.where(wid < _EXTRA, _BASE_BLKS + 1, _BASE_BLKS)
        blk0 = _BASE_BLKS * wid + jnp.minimum(wid, _EXTRA)
        pairs = nblk // 2

        def start_in(blk, cen_v, nei_v, eng_v, sem):
            pltpu.async_copy(ei_hbm.at[0, blk], cen_v, sem)
            pltpu.async_copy(ei_hbm.at[1, blk], nei_v, sem)
            pltpu.async_copy(eng_hbm.at[blk], eng_v, sem)

        def wait_in(cen_v, nei_v, eng_v, sem):
            pltpu.make_async_copy(ei_hbm.at[0, 0], cen_v, sem).wait()
            pltpu.make_async_copy(ei_hbm.at[1, 0], nei_v, sem).wait()
            pltpu.make_async_copy(eng_hbm.at[0], eng_v, sem).wait()

        def compute(cen_v, nei_v, eng_v, val_v):
            def sub_body(i, _):
                j = i // (_CHUNK // _LANES)
                s = (i % (_CHUNK // _LANES)) * _LANES
                c = cen_v[j, pl.ds(s, _LANES)]
                n = nei_v[j, pl.ds(s, _LANES)]
                cs = plsc.load_gather(species_v, [c])
                ns = plsc.load_gather(species_v, [n])
                scale = plsc.load_gather(table_v, [cs * _NUM_TYPES + ns])
                val_v[j, pl.ds(s, _LANES)] = eng_v[j, pl.ds(s, _LANES)] * scale
                return 0

            lax.fori_loop(0, _BLK // _LANES, sub_body, 0)

        def fire_scat(cen_v, val_v, sem):
            for j in range(_ROWS):
                pltpu.async_copy(
                    val_v.at[j], acc_sh.at[cen_v.at[j]], sem, add=True)

        def drain_scat(cen_v, val_v, sem):
            for j in range(_ROWS):
                pltpu.make_async_copy(
                    val_v.at[j], acc_sh.at[cen_v.at[j]], sem).wait()

        start_in(blk0, cen_a, nei_a, eng_a, in_sem_a)

        def pair_body(p, _):
            blk = blk0 + 2 * p
            wait_in(cen_a, nei_a, eng_a, in_sem_a)
            compute(cen_a, nei_a, eng_a, val_a)
            fire_scat(cen_a, val_a, scat_sem_a)

            @pl.when(p > 0)
            def _():
                drain_scat(cen_b, val_b, scat_sem_b)

            start_in(blk + 1, cen_b, nei_b, eng_b, in_sem_b)

            wait_in(cen_b, nei_b, eng_b, in_sem_b)
            compute(cen_b, nei_b, eng_b, val_b)
            drain_scat(cen_a, val_a, scat_sem_a)
            fire_scat(cen_b, val_b, scat_sem_b)

            @pl.when(2 * p + 2 < nblk)
            def _():
                start_in(blk + 2, cen_a, nei_a, eng_a, in_sem_a)

            return 0

        lax.fori_loop(0, pairs, pair_body, 0)

        @pl.when(nblk % 2 == 1)
        def _():
            wait_in(cen_a, nei_a, eng_a, in_sem_a)
            compute(cen_a, nei_a, eng_a, val_a)
            fire_scat(cen_a, val_a, scat_sem_a)
            drain_scat(cen_a, val_a, scat_sem_a)

        drain_scat(cen_b, val_b, scat_sem_b)
        plsc.subcore_barrier()

        sl = pl.ds(tid * _TILE_SLICE, _TILE_SLICE)
        pltpu.sync_copy(acc_sh.at[sl], stage_v)
        pltpu.sync_copy(stage_v, out_hbm.at[cid, sl])

    return k(eng3, ei4, species, table16)


def _tc_add(partials):
    def body(p_ref, o_ref):
        o_ref[...] = p_ref[0] + p_ref[1]

    return pl.pallas_call(
        body,
        out_shape=jax.ShapeDtypeStruct((_ACC_PAD // 128, 128), jnp.float32),
    )(partials.reshape(2, _ACC_PAD // 128, 128))


def kernel(edge_energy, per_edge_scales, edge_index, atom_types):
    eng3 = edge_energy.reshape(_NBLK, _ROWS, _CHUNK)
    ei4 = edge_index.reshape(2, _NBLK, _ROWS, _CHUNK)
    species = atom_types.reshape(_N_NODES)
    table16 = per_edge_scales.reshape(_NUM_TYPES * _NUM_TYPES)

    partials = _sc_partial_sums(eng3, ei4, species, table16)
    summed = _tc_add(partials)
    return summed.reshape(_ACC_PAD)[:_N_NODES].reshape(_N_NODES, 1)
